# baseline (speedup 1.0000x reference)
import functools

import jax
import jax.numpy as jnp
from jax import lax
from jax.experimental import pallas as pl
from jax.experimental.pallas import tpu as pltpu
from jax.experimental.pallas import tpu_sc as plsc

N_MODS = 16
EMBED_DIM = 64
BATCH = 16384

_info = plsc.get_sparse_core_info()
_NC, _NS, _L = _info.num_cores, _info.num_subcores, _info.num_lanes
_NW = _NC * _NS
_BPW = BATCH // _NW
_RPB = 16
_NPASS = 2
_RPP = _BPW // _NPASS
_NBATCH = _RPP // _RPB
_LAG = 2
_GROUPS = _RPP // _L


def _sc_body(pidx_hbm, bidx_hbm, mbits_hbm, ptab_hbm, mtab_hbm, out_hbm,
             bidx_s, mbits_s, kidx_v, pidx_v,
             pidx_sm, kidx_sm,
             prow, mrow, partials, out_v, sem_p, sem_m):
    wid = lax.axis_index("s") * _NC + lax.axis_index("c")
    base = wid * _BPW

    pltpu.sync_copy(pidx_hbm.at[pl.ds(base, _BPW)], pidx_v)
    pltpu.sync_copy(bidx_hbm.at[pl.ds(base, _BPW)], bidx_s)
    pltpu.sync_copy(mbits_hbm.at[pl.ds(base, _BPW)], mbits_s)

    def stage_body(i, _):
        sl = pl.ds(i * _L, _L)
        pv = pidx_v[sl]
        kv = bidx_s[sl] * N_MODS + mbits_s[sl]
        ib = i * _L
        for lane in range(_L):
            pidx_sm[ib + lane] = pv[lane]
            kidx_sm[ib + lane] = kv[lane]
        return 0

    lax.fori_loop(0, _BPW // _L, stage_body, 0)

    iota = lax.iota(jnp.int32, _L)
    colbase = iota * _L

    for p in range(_NPASS):
        off = p * _RPP

        def batch_body(b, _, off=off):
            @pl.when(b < _NBATCH)
            def _fire():
                ib = b * _RPB
                for j in range(_RPB):
                    r = pidx_sm[off + ib + j]
                    pltpu.make_async_copy(ptab_hbm.at[r], prow.at[ib + j],
                                          sem_p).start()
                    k = kidx_sm[off + ib + j]
                    pltpu.make_async_copy(mtab_hbm.at[k], mrow.at[ib + j],
                                          sem_m).start()

            @pl.when(b >= _LAG)
            def _drain():
                ib = (b - _LAG) * _RPB
                for j in range(_RPB):
                    pltpu.make_async_copy(ptab_hbm.at[0], prow.at[ib + j],
                                          sem_p).wait()
                    pltpu.make_async_copy(mtab_hbm.at[0], mrow.at[ib + j],
                                          sem_m).wait()
            return 0

        lax.fori_loop(0, _NBATCH + _LAG, batch_body, 0)

        def group_body(g, _, off=off):
            for r in range(_L):
                rr = g * _L + r
                part = prow[rr, pl.ds(0, _L)] * mrow[rr, pl.ds(0, _L)]
                for k in range(1, EMBED_DIM // _L):
                    sl = pl.ds(k * _L, _L)
                    part = part + prow[rr, sl] * mrow[rr, sl]
                partials[pl.ds(r * _L, _L)] = part
            acc = plsc.load_gather(partials, [colbase])
            for l in range(1, _L):
                acc = acc + plsc.load_gather(partials, [colbase + l])
            out_v[pl.ds(off + g * _L, _L)] = acc
            return 0

        lax.fori_loop(0, _GROUPS, group_body, 0)

    pltpu.sync_copy(out_v, out_hbm.at[pl.ds(base, _BPW)])


@jax.jit
def _run(player_indices, beatmap_ids, mod_bits, player_table, map_table):
    mesh = plsc.VectorSubcoreMesh(core_axis_name="c", subcore_axis_name="s")
    f = functools.partial(
        pl.kernel,
        out_type=jax.ShapeDtypeStruct((BATCH,), jnp.float32),
        mesh=mesh,
        compiler_params=pltpu.CompilerParams(needs_layout_passes=False),
        scratch_types=[
            pltpu.VMEM((_BPW,), jnp.int32),
            pltpu.VMEM((_BPW,), jnp.int32),
            pltpu.VMEM((_BPW,), jnp.int32),
            pltpu.VMEM((_BPW,), jnp.int32),
            pltpu.SMEM((_BPW,), jnp.int32),
            pltpu.SMEM((_BPW,), jnp.int32),
            pltpu.VMEM((_RPP, EMBED_DIM), jnp.float32),
            pltpu.VMEM((_RPP, EMBED_DIM), jnp.float32),
            pltpu.VMEM((_L * _L,), jnp.float32),
            pltpu.VMEM((_BPW,), jnp.float32),
            pltpu.SemaphoreType.DMA,
            pltpu.SemaphoreType.DMA,
        ],
    )(_sc_body)
    return f(player_indices, beatmap_ids, mod_bits, player_table, map_table)


def kernel(player_indices, beatmap_ids, mod_bits, player_table, map_table):
    return _run(player_indices.astype(jnp.int32),
                beatmap_ids.astype(jnp.int32),
                mod_bits.astype(jnp.int32),
                player_table, map_table)
